# baseline (speedup 1.0000x reference)
import jax
import jax.numpy as jnp
from jax.experimental import pallas as pl

D_MODEL = 768
NUM_EXPERTS = 64
TOKEN_BLOCK = 512
N_STREAMS = 16


def _router_body(*refs):
    w = refs[N_STREAMS][...]
    T = TOKEN_BLOCK
    for s in range(N_STREAMS):
        x = refs[s][...]
        g_ref = refs[N_STREAMS + 1 + 2 * s]
        i_ref = refs[N_STREAMS + 2 + 2 * s]
        logits = jax.lax.dot_general(
            x, w, (((1,), (1,)), ((), ())),
            preferred_element_type=jnp.float32)
        i1 = jnp.argmax(logits, axis=-1)
        m1 = jnp.max(logits, axis=-1)
        iota = jax.lax.broadcasted_iota(jnp.int32, logits.shape, 1)
        masked = jnp.where(iota == i1[:, None], -jnp.inf, logits)
        i2 = jnp.argmax(masked, axis=-1)
        m2 = jnp.max(masked, axis=-1)
        g1 = 1.0 / (1.0 + jnp.exp(m2 - m1))
        g_ref[...] = jnp.stack([g1, 1.0 - g1], axis=0)
        i_ref[...] = jnp.stack([i1, i2], axis=0).astype(jnp.int32)


def kernel(x, W):
    B, S, D = x.shape
    N = B * S
    xf = x.reshape(N, D)
    T = TOKEN_BLOCK
    ns = N_STREAMS
    G = N // (T * ns)
    grid = (G,)

    def x_spec(s):
        return pl.BlockSpec((T, D), lambda i, s=s: (s * G + i, 0))

    outs = pl.pallas_call(
        _router_body,
        grid=grid,
        in_specs=[x_spec(s) for s in range(ns)]
        + [pl.BlockSpec((NUM_EXPERTS, D), lambda i: (0, 0))],
        out_specs=[spec for s in range(ns)
                   for spec in (pl.BlockSpec((2, T), lambda i: (0, i)),
                                pl.BlockSpec((2, T), lambda i: (0, i)))],
        out_shape=[shape for _ in range(ns)
                   for shape in (jax.ShapeDtypeStruct((2, G * T), jnp.float32),
                                 jax.ShapeDtypeStruct((2, G * T), jnp.int32))],
    )(*([xf] * ns), W)
    gates_t = jnp.concatenate(outs[0::2], axis=1)
    indices_t = jnp.concatenate(outs[1::2], axis=1)
    gates = gates_t.T.reshape(B, S, 2)
    indices = indices_t.T.reshape(B, S, 2)
    return gates, indices
